# Initial kernel scaffold; baseline (speedup 1.0000x reference)
import functools
import jax
import jax.numpy as jnp
from jax import lax
from jax.experimental import pallas as pl
from jax.experimental.pallas import tpu as pltpu
import jax.experimental.pallas.tpu_sc as plsc

N = 10000
E = 160000
NC, NS = 2, 16
NPAD = 10240
ROWS_PER_TILE = NPAD // NS
B = 128
NB = 79
EPAD = NS * NB * B
DEG_NB = 40
DEG_EPAD_HALF = NS * DEG_NB * B
SINK = 10016

_mesh = plsc.VectorSubcoreMesh(
    core_axis_name="c", subcore_axis_name="s", num_cores=NC, num_subcores=NS
)


def _fill(ref, rows, value):
    vec = jnp.full((16,), value, jnp.float32)

    def body(i, _):
        for k in range(8):
            ref[i, pl.ds(k * 16, 16)] = vec
        return 0

    lax.fori_loop(0, rows, body, 0)


@functools.partial(
    pl.kernel,
    out_type=jax.ShapeDtypeStruct((NC, NPAD), jnp.float32),
    mesh=_mesh,
    scratch_types=[
        pltpu.VMEM((DEG_NB, B), jnp.int32),
        pltpu.VMEM((B,), jnp.float32),
        pltpu.VMEM((ROWS_PER_TILE,), jnp.float32),
        pltpu.VMEM_SHARED((NPAD,), jnp.float32),
    ],
)
def _deg_kernel(dstd, out, dv, ones_v, zb, acc):
    c = lax.axis_index("c")
    s = lax.axis_index("s")
    one16 = jnp.ones((16,), jnp.float32)
    zero16 = jnp.zeros((16,), jnp.float32)
    for k in range(B // 16):
        ones_v[pl.ds(k * 16, 16)] = one16
    for k in range(ROWS_PER_TILE // 16):
        zb[pl.ds(k * 16, 16)] = zero16
    pltpu.sync_copy(zb, acc.at[pl.ds(s * ROWS_PER_TILE, ROWS_PER_TILE)])
    pltpu.sync_copy(dstd.at[c, s], dv)
    plsc.subcore_barrier()

    def blk(j, _):
        pltpu.sync_copy(ones_v, acc.at[dv.at[j]], add=True)
        return 0

    lax.fori_loop(0, DEG_NB, blk, 0)
    plsc.subcore_barrier()
    pltpu.sync_copy(
        acc.at[pl.ds(s * ROWS_PER_TILE, ROWS_PER_TILE)],
        out.at[c, pl.ds(s * ROWS_PER_TILE, ROWS_PER_TILE)],
    )


def _make_spmm(C):
    CPC = C // NC

    @functools.partial(
        pl.kernel,
        out_type=jax.ShapeDtypeStruct((C, NPAD, 128), jnp.float32),
        mesh=_mesh,
        scratch_types=[
            pltpu.VMEM((NB, B), jnp.int32),
            pltpu.VMEM((NB, B), jnp.int32),
            pltpu.VMEM((B, 128), jnp.float32),
            pltpu.VMEM((B, 128), jnp.float32),
            pltpu.VMEM_SHARED((NPAD, 128), jnp.float32),
        ],
    )
    def spmm(htab, srcp, dstp, out, sv, dv, rows, zb, acc):
        c = lax.axis_index("c")
        s = lax.axis_index("s")
        _fill(zb, B, 0.0)
        pltpu.sync_copy(dstp.at[s], dv)

        for cc in range(CPC):
            chunk = c * CPC + cc
            pltpu.sync_copy(srcp.at[s], sv)
            off = (chunk * N).astype(jnp.int32)

            def shift(i, _):
                for k in range(B // 16):
                    sl = pl.ds(k * 16, 16)
                    sv[i, sl] = sv[i, sl] + off
                return 0

            lax.fori_loop(0, NB, shift, 0)

            for z in range(ROWS_PER_TILE // B):
                pltpu.sync_copy(
                    zb, acc.at[pl.ds(s * ROWS_PER_TILE + z * B, B)]
                )
            plsc.subcore_barrier()

            def blk(j, _):
                pltpu.sync_copy(htab.at[sv.at[j]], rows)
                pltpu.sync_copy(rows, acc.at[dv.at[j]], add=True)
                return 0

            lax.fori_loop(0, NB, blk, 0)
            plsc.subcore_barrier()
            pltpu.sync_copy(
                acc.at[pl.ds(s * ROWS_PER_TILE, ROWS_PER_TILE)],
                out.at[chunk, pl.ds(s * ROWS_PER_TILE, ROWS_PER_TILE)],
            )
            plsc.subcore_barrier()

    return spmm


_spmm2 = _make_spmm(2)
_spmm4 = _make_spmm(4)


def _dinv_of(degT_blk):
    dsum = degT_blk[:, 0:1] + degT_blk[:, 1:2] + 1.0
    return lax.rsqrt(dsum)


def _k1_body(x_ref, w_ref, degT_ref, out_ref):
    dinv = _dinv_of(degT_ref[...])
    h = jnp.dot(x_ref[...], w_ref[0], preferred_element_type=jnp.float32)
    out_ref[0] = h * dinv


def _k2_body(msg_ref, hp_ref, degT_ref, b_ref, w_ref, out_ref):
    dinv = _dinv_of(degT_ref[...])
    parts = []
    for j in range(2):
        t = dinv * (msg_ref[j] + hp_ref[j]) + b_ref[j][None, :]
        parts.append(jnp.maximum(t, 0.0))
    o1 = jnp.concatenate(parts, axis=1)
    h = jnp.dot(o1, w_ref[0], preferred_element_type=jnp.float32)
    out_ref[0] = h * dinv


def _k3_body(msg_ref, hp_ref, degT_ref, b_ref, out_ref):
    dinv = _dinv_of(degT_ref[...])
    parts = []
    for j in range(4):
        parts.append(dinv * (msg_ref[j] + hp_ref[j]) + b_ref[j][None, :])
    out_ref[...] = jnp.concatenate(parts, axis=1)


RB = 400


def kernel(x, edge_index, W1, b1, W2, b2):
    ei = edge_index.astype(jnp.int32)
    src, dst = ei[0], ei[1]

    pad = EPAD - E
    srcp = jnp.concatenate([src, jnp.zeros((pad,), jnp.int32)]).reshape(NS, NB, B)
    dstp = jnp.concatenate([dst, jnp.full((pad,), SINK, jnp.int32)]).reshape(NS, NB, B)
    half = E // 2
    hpad = DEG_EPAD_HALF - half
    dstd = jnp.concatenate(
        [
            dst[:half], jnp.full((hpad,), SINK, jnp.int32),
            dst[half:], jnp.full((hpad,), SINK, jnp.int32),
        ]
    ).reshape(NC, NS, DEG_NB, B)

    W1r = W1.reshape(512, 2, 128).transpose(1, 0, 2)
    W2r = W2.reshape(256, 4, 128).transpose(1, 0, 2)
    b1r = b1.reshape(2, 128)
    b2r = b2.reshape(4, 128)

    degp = _deg_kernel(dstd)
    degT = degp.T[:N]

    h1p = pl.pallas_call(
        _k1_body,
        grid=(25, 2),
        in_specs=[
            pl.BlockSpec((RB, 512), lambda i, c: (i, 0)),
            pl.BlockSpec((1, 512, 128), lambda i, c: (c, 0, 0)),
            pl.BlockSpec((RB, 2), lambda i, c: (i, 0)),
        ],
        out_specs=pl.BlockSpec((1, RB, 128), lambda i, c: (c, i, 0)),
        out_shape=jax.ShapeDtypeStruct((2, N, 128), jnp.float32),
    )(x, W1r, degT)

    msg1 = _spmm2(h1p.reshape(2 * N, 128), srcp, dstp)[:, :N, :]

    h2p = pl.pallas_call(
        _k2_body,
        grid=(25, 4),
        in_specs=[
            pl.BlockSpec((2, RB, 128), lambda i, c: (0, i, 0)),
            pl.BlockSpec((2, RB, 128), lambda i, c: (0, i, 0)),
            pl.BlockSpec((RB, 2), lambda i, c: (i, 0)),
            pl.BlockSpec((2, 128), lambda i, c: (0, 0)),
            pl.BlockSpec((1, 256, 128), lambda i, c: (c, 0, 0)),
        ],
        out_specs=pl.BlockSpec((1, RB, 128), lambda i, c: (c, i, 0)),
        out_shape=jax.ShapeDtypeStruct((4, N, 128), jnp.float32),
    )(msg1, h1p, degT, b1r, W2r)

    msg2 = _spmm4(h2p.reshape(4 * N, 128), srcp, dstp)[:, :N, :]

    out = pl.pallas_call(
        _k3_body,
        grid=(25,),
        in_specs=[
            pl.BlockSpec((4, RB, 128), lambda i: (0, i, 0)),
            pl.BlockSpec((4, RB, 128), lambda i: (0, i, 0)),
            pl.BlockSpec((RB, 2), lambda i: (i, 0)),
            pl.BlockSpec((4, 128), lambda i: (0, 0)),
        ],
        out_specs=pl.BlockSpec((RB, 512), lambda i: (i, 0)),
        out_shape=jax.ShapeDtypeStruct((N, 512), jnp.float32),
    )(msg2, h2p, degT, b2r)
    return out


# speedup vs baseline: 3.8505x; 3.8505x over previous
import functools
import jax
import jax.numpy as jnp
from jax import lax
from jax.experimental import pallas as pl
from jax.experimental.pallas import tpu as pltpu
import jax.experimental.pallas.tpu_sc as plsc

N = 10000
E = 160000
NC, NS = 2, 16
HALF = N // NC
HPAD = 5120
ROWS_PER_TILE = HPAD // NS
LSINK = 5056
B = 128
NB = 79
EPAD = NS * NB * B
NPAD = 10240
DROWS = NPAD // NS
DEG_NB = 40
DEG_EPAD_HALF = NS * DEG_NB * B
DSINK = 10016

_mesh = plsc.VectorSubcoreMesh(
    core_axis_name="c", subcore_axis_name="s", num_cores=NC, num_subcores=NS
)


@functools.partial(
    pl.kernel,
    out_type=jax.ShapeDtypeStruct((NC, NPAD), jnp.float32),
    mesh=_mesh,
    scratch_types=[
        pltpu.VMEM((DEG_NB, B), jnp.int32),
        pltpu.VMEM((B,), jnp.float32),
        pltpu.VMEM((DROWS,), jnp.float32),
        pltpu.VMEM_SHARED((NPAD,), jnp.float32),
    ],
)
def _deg_kernel(dstd, out, dv, ones_v, zb, acc):
    c = lax.axis_index("c")
    s = lax.axis_index("s")
    one16 = jnp.ones((16,), jnp.float32)
    zero16 = jnp.zeros((16,), jnp.float32)
    for k in range(B // 16):
        ones_v[pl.ds(k * 16, 16)] = one16
    for k in range(DROWS // 16):
        zb[pl.ds(k * 16, 16)] = zero16
    pltpu.sync_copy(zb, acc.at[pl.ds(s * DROWS, DROWS)])
    pltpu.sync_copy(dstd.at[c, s], dv)
    plsc.subcore_barrier()

    def blk(j, _):
        pltpu.sync_copy(ones_v, acc.at[dv.at[j]], add=True)
        return 0

    lax.fori_loop(0, DEG_NB, blk, 0)
    plsc.subcore_barrier()
    pltpu.sync_copy(
        acc.at[pl.ds(s * DROWS, DROWS)],
        out.at[c, pl.ds(s * DROWS, DROWS)],
    )


def _make_spmm(C):
    @functools.partial(
        pl.kernel,
        out_type=jax.ShapeDtypeStruct((C, NC, HPAD, 128), jnp.float32),
        mesh=_mesh,
        scratch_types=[
            pltpu.VMEM((NB, B), jnp.int32),
            pltpu.VMEM((NB, B), jnp.int32),
            pltpu.VMEM((B, 128), jnp.float32),
            pltpu.VMEM((B, 128), jnp.float32),
            pltpu.VMEM_SHARED((HPAD, 128), jnp.float32),
        ],
    )
    def spmm(htab, srcp, dstp, out, sv, dv, rows, zb, acc):
        c = lax.axis_index("c")
        s = lax.axis_index("s")
        zero16 = jnp.zeros((16,), jnp.float32)

        def zrow(i, _):
            for k in range(8):
                zb[i, pl.ds(k * 16, 16)] = zero16
            return 0

        lax.fori_loop(0, B, zrow, 0)

        pltpu.sync_copy(dstp.at[s], dv)
        lo = (c * HALF).astype(jnp.int32)
        sink16 = jnp.full((16,), LSINK, jnp.int32)

        def remap(i, _):
            for k in range(B // 16):
                sl = pl.ds(k * 16, 16)
                d = dv[i, sl]
                dl = d - lo
                ok = (dl >= 0) & (dl < HALF)
                dv[i, sl] = jnp.where(ok, dl, sink16)
            return 0

        lax.fori_loop(0, NB, remap, 0)

        for chunk in range(C):
            pltpu.sync_copy(srcp.at[s], sv)
            off = jnp.int32(chunk * N)

            def shift(i, _):
                for k in range(B // 16):
                    sl = pl.ds(k * 16, 16)
                    sv[i, sl] = sv[i, sl] + off
                return 0

            if chunk:
                lax.fori_loop(0, NB, shift, 0)

            for z in range(ROWS_PER_TILE // B):
                pltpu.sync_copy(zb, acc.at[pl.ds(s * ROWS_PER_TILE + z * B, B)])
            rem = ROWS_PER_TILE % B
            if rem:
                pltpu.sync_copy(
                    zb.at[pl.ds(0, rem)],
                    acc.at[pl.ds(s * ROWS_PER_TILE + ROWS_PER_TILE - rem, rem)],
                )
            plsc.subcore_barrier()

            def blk(j, _):
                pltpu.sync_copy(htab.at[sv.at[j]], rows)
                pltpu.sync_copy(rows, acc.at[dv.at[j]], add=True)
                return 0

            lax.fori_loop(0, NB, blk, 0)
            plsc.subcore_barrier()
            for z in range(ROWS_PER_TILE // B):
                pltpu.sync_copy(
                    acc.at[pl.ds(s * ROWS_PER_TILE + z * B, B)],
                    out.at[chunk, c, pl.ds(s * ROWS_PER_TILE + z * B, B)],
                )
            if rem:
                pltpu.sync_copy(
                    acc.at[pl.ds(s * ROWS_PER_TILE + ROWS_PER_TILE - rem, rem)],
                    out.at[chunk, c, pl.ds(s * ROWS_PER_TILE + ROWS_PER_TILE - rem, rem)],
                )
            plsc.subcore_barrier()

    return spmm


_spmm2 = _make_spmm(2)
_spmm4 = _make_spmm(4)


def _dinv_of(degT_blk):
    dsum = degT_blk[:, 0:1] + degT_blk[:, 1:2] + 1.0
    return lax.rsqrt(dsum)


def _k1_body(x_ref, w_ref, degT_ref, out_ref):
    dinv = _dinv_of(degT_ref[...])
    h = jnp.dot(x_ref[...], w_ref[0], preferred_element_type=jnp.float32)
    out_ref[0] = h * dinv


def _k2_body(msg_ref, hp_ref, degT_ref, b_ref, w_ref, out_ref):
    dinv = _dinv_of(degT_ref[...])
    parts = []
    for j in range(2):
        t = dinv * (msg_ref[j] + hp_ref[j]) + b_ref[j][None, :]
        parts.append(jnp.maximum(t, 0.0))
    o1 = jnp.concatenate(parts, axis=1)
    h = jnp.dot(o1, w_ref[0], preferred_element_type=jnp.float32)
    out_ref[0] = h * dinv


def _k3_body(msg_ref, hp_ref, degT_ref, b_ref, out_ref):
    dinv = _dinv_of(degT_ref[...])
    parts = []
    for j in range(4):
        parts.append(dinv * (msg_ref[j] + hp_ref[j]) + b_ref[j][None, :])
    out_ref[...] = jnp.concatenate(parts, axis=1)


RB = 400


def _assemble(msgp, C):
    return jnp.concatenate([msgp[:, 0, :HALF], msgp[:, 1, :HALF]], axis=1)


def kernel(x, edge_index, W1, b1, W2, b2):
    ei = edge_index.astype(jnp.int32)
    src, dst = ei[0], ei[1]

    pad = EPAD - E
    srcp = jnp.concatenate([src, jnp.zeros((pad,), jnp.int32)]).reshape(NS, NB, B)
    dstp = jnp.concatenate([dst, jnp.full((pad,), DSINK, jnp.int32)]).reshape(NS, NB, B)
    half = E // 2
    hpad = DEG_EPAD_HALF - half
    dstd = jnp.concatenate(
        [
            dst[:half], jnp.full((hpad,), DSINK, jnp.int32),
            dst[half:], jnp.full((hpad,), DSINK, jnp.int32),
        ]
    ).reshape(NC, NS, DEG_NB, B)

    W1r = W1.reshape(512, 2, 128).transpose(1, 0, 2)
    W2r = W2.reshape(256, 4, 128).transpose(1, 0, 2)
    b1r = b1.reshape(2, 128)
    b2r = b2.reshape(4, 128)

    degp = _deg_kernel(dstd)
    degT = degp.T[:N]

    h1p = pl.pallas_call(
        _k1_body,
        grid=(25, 2),
        in_specs=[
            pl.BlockSpec((RB, 512), lambda i, c: (i, 0)),
            pl.BlockSpec((1, 512, 128), lambda i, c: (c, 0, 0)),
            pl.BlockSpec((RB, 2), lambda i, c: (i, 0)),
        ],
        out_specs=pl.BlockSpec((1, RB, 128), lambda i, c: (c, i, 0)),
        out_shape=jax.ShapeDtypeStruct((2, N, 128), jnp.float32),
    )(x, W1r, degT)

    msg1 = _assemble(_spmm2(h1p.reshape(2 * N, 128), srcp, dstp), 2)

    h2p = pl.pallas_call(
        _k2_body,
        grid=(25, 4),
        in_specs=[
            pl.BlockSpec((2, RB, 128), lambda i, c: (0, i, 0)),
            pl.BlockSpec((2, RB, 128), lambda i, c: (0, i, 0)),
            pl.BlockSpec((RB, 2), lambda i, c: (i, 0)),
            pl.BlockSpec((2, 128), lambda i, c: (0, 0)),
            pl.BlockSpec((1, 256, 128), lambda i, c: (c, 0, 0)),
        ],
        out_specs=pl.BlockSpec((1, RB, 128), lambda i, c: (c, i, 0)),
        out_shape=jax.ShapeDtypeStruct((4, N, 128), jnp.float32),
    )(msg1, h1p, degT, b1r, W2r)

    msg2 = _assemble(_spmm4(h2p.reshape(4 * N, 128), srcp, dstp), 4)

    out = pl.pallas_call(
        _k3_body,
        grid=(25,),
        in_specs=[
            pl.BlockSpec((4, RB, 128), lambda i: (0, i, 0)),
            pl.BlockSpec((4, RB, 128), lambda i: (0, i, 0)),
            pl.BlockSpec((RB, 2), lambda i: (i, 0)),
            pl.BlockSpec((4, 128), lambda i: (0, 0)),
        ],
        out_specs=pl.BlockSpec((RB, 512), lambda i: (i, 0)),
        out_shape=jax.ShapeDtypeStruct((N, 512), jnp.float32),
    )(msg2, h2p, degT, b2r)
    return out


# speedup vs baseline: 3.8632x; 1.0033x over previous
import functools
import jax
import jax.numpy as jnp
from jax import lax
from jax.experimental import pallas as pl
from jax.experimental.pallas import tpu as pltpu
import jax.experimental.pallas.tpu_sc as plsc

N = 10000
E = 160000
NC, NS = 2, 16
HALF = N // NC
HPAD = 5120
ROWS_PER_TILE = HPAD // NS
LSINK = 5056
B = 128
NB = 79
EPAD = NS * NB * B
NPAD = 10240
DROWS = NPAD // NS
DEG_NB = 40
DEG_EPAD_HALF = NS * DEG_NB * B
DSINK = 10016

_mesh = plsc.VectorSubcoreMesh(
    core_axis_name="c", subcore_axis_name="s", num_cores=NC, num_subcores=NS
)


@functools.partial(
    pl.kernel,
    out_type=jax.ShapeDtypeStruct((NC, NPAD), jnp.float32),
    mesh=_mesh,
    scratch_types=[
        pltpu.VMEM((DEG_NB, B), jnp.int32),
        pltpu.VMEM((B,), jnp.float32),
        pltpu.VMEM((DROWS,), jnp.float32),
        pltpu.VMEM_SHARED((NPAD,), jnp.float32),
    ],
)
def _deg_kernel(dstd, out, dv, ones_v, zb, acc):
    c = lax.axis_index("c")
    s = lax.axis_index("s")
    one16 = jnp.ones((16,), jnp.float32)
    zero16 = jnp.zeros((16,), jnp.float32)
    for k in range(B // 16):
        ones_v[pl.ds(k * 16, 16)] = one16
    for k in range(DROWS // 16):
        zb[pl.ds(k * 16, 16)] = zero16
    pltpu.sync_copy(zb, acc.at[pl.ds(s * DROWS, DROWS)])
    pltpu.sync_copy(dstd.at[c, s], dv)
    plsc.subcore_barrier()

    def blk(j, _):
        pltpu.sync_copy(ones_v, acc.at[dv.at[j]], add=True)
        return 0

    lax.fori_loop(0, DEG_NB, blk, 0)
    plsc.subcore_barrier()
    pltpu.sync_copy(
        acc.at[pl.ds(s * DROWS, DROWS)],
        out.at[c, pl.ds(s * DROWS, DROWS)],
    )


def _make_spmm(C):
    @functools.partial(
        pl.kernel,
        out_type=jax.ShapeDtypeStruct((C, NC, HPAD, 128), jnp.float32),
        mesh=_mesh,
        scratch_types=[
            pltpu.VMEM((NB, B), jnp.int32),
            pltpu.VMEM((NB, B), jnp.int32),
            pltpu.VMEM((B, 128), jnp.float32),
            pltpu.VMEM((64, 128), jnp.float32),
            pltpu.VMEM_SHARED((HPAD, 128), jnp.float32),
        ],
    )
    def spmm(htab, srcp, dstp, out, sv, dv, rows, zb, acc):
        c = lax.axis_index("c")
        s = lax.axis_index("s")
        zero16 = jnp.zeros((16,), jnp.float32)

        def zrow(i, _):
            for k in range(8):
                zb[i, pl.ds(k * 16, 16)] = zero16
            return 0

        lax.fori_loop(0, 64, zrow, 0)

        pltpu.sync_copy(dstp.at[s], dv)
        lo = (c * HALF).astype(jnp.int32)
        sink16 = jnp.full((16,), LSINK, jnp.int32)

        def remap(i, _):
            for k in range(B // 16):
                sl = pl.ds(k * 16, 16)
                d = dv[i, sl]
                dl = d - lo
                ok = (dl >= 0) & (dl < HALF)
                dv[i, sl] = jnp.where(ok, dl, sink16)
            return 0

        lax.fori_loop(0, NB, remap, 0)

        for chunk in range(C):
            pltpu.sync_copy(srcp.at[s], sv)
            off = jnp.int32(chunk * N)

            def shift(i, _):
                for k in range(B // 16):
                    sl = pl.ds(k * 16, 16)
                    sv[i, sl] = sv[i, sl] + off
                return 0

            if chunk:
                lax.fori_loop(0, NB, shift, 0)

            for z in range(ROWS_PER_TILE // 64):
                pltpu.sync_copy(zb, acc.at[pl.ds(s * ROWS_PER_TILE + z * 64, 64)])
            plsc.subcore_barrier()

            def blk(j, _):
                pltpu.sync_copy(htab.at[sv.at[j]], rows)
                pltpu.sync_copy(rows, acc.at[dv.at[j]], add=True)
                return 0

            lax.fori_loop(0, NB, blk, 0)
            plsc.subcore_barrier()
            for z in range(ROWS_PER_TILE // 64):
                pltpu.sync_copy(
                    acc.at[pl.ds(s * ROWS_PER_TILE + z * 64, 64)],
                    out.at[chunk, c, pl.ds(s * ROWS_PER_TILE + z * 64, 64)],
                )
            plsc.subcore_barrier()

    return spmm


_spmm2 = _make_spmm(2)
_spmm4 = _make_spmm(4)


def _dinv_of(degT_blk):
    dsum = degT_blk[:, 0:1] + degT_blk[:, 1:2] + 1.0
    return lax.rsqrt(dsum)


def _k1_body(x_ref, w_ref, degT_ref, out_ref):
    dinv = _dinv_of(degT_ref[...])
    h = jnp.dot(x_ref[...], w_ref[0], preferred_element_type=jnp.float32)
    out_ref[0] = h * dinv


def _k2_body(msg_ref, hp_ref, degT_ref, b_ref, w_ref, out_ref):
    dinv = _dinv_of(degT_ref[...])
    parts = []
    for j in range(2):
        t = dinv * (msg_ref[j] + hp_ref[j]) + b_ref[j][None, :]
        parts.append(jnp.maximum(t, 0.0))
    o1 = jnp.concatenate(parts, axis=1)
    h = jnp.dot(o1, w_ref[0], preferred_element_type=jnp.float32)
    out_ref[0] = h * dinv


def _k3_body(msg_ref, hp_ref, degT_ref, b_ref, out_ref):
    dinv = _dinv_of(degT_ref[...])
    parts = []
    for j in range(4):
        parts.append(dinv * (msg_ref[j] + hp_ref[j]) + b_ref[j][None, :])
    out_ref[...] = jnp.concatenate(parts, axis=1)


RB = 400


def _assemble(msgp, C):
    return jnp.concatenate([msgp[:, 0, :HALF], msgp[:, 1, :HALF]], axis=1)


def kernel(x, edge_index, W1, b1, W2, b2):
    ei = edge_index.astype(jnp.int32)
    src, dst = ei[0], ei[1]

    pad = EPAD - E
    srcp = jnp.concatenate([src, jnp.zeros((pad,), jnp.int32)]).reshape(NS, NB, B)
    dstp = jnp.concatenate([dst, jnp.full((pad,), DSINK, jnp.int32)]).reshape(NS, NB, B)
    half = E // 2
    hpad = DEG_EPAD_HALF - half
    dstd = jnp.concatenate(
        [
            dst[:half], jnp.full((hpad,), DSINK, jnp.int32),
            dst[half:], jnp.full((hpad,), DSINK, jnp.int32),
        ]
    ).reshape(NC, NS, DEG_NB, B)

    W1r = W1.reshape(512, 2, 128).transpose(1, 0, 2)
    W2r = W2.reshape(256, 4, 128).transpose(1, 0, 2)
    b1r = b1.reshape(2, 128)
    b2r = b2.reshape(4, 128)

    degp = _deg_kernel(dstd)
    degT = degp.T[:N]

    h1p = pl.pallas_call(
        _k1_body,
        grid=(25, 2),
        in_specs=[
            pl.BlockSpec((RB, 512), lambda i, c: (i, 0)),
            pl.BlockSpec((1, 512, 128), lambda i, c: (c, 0, 0)),
            pl.BlockSpec((RB, 2), lambda i, c: (i, 0)),
        ],
        out_specs=pl.BlockSpec((1, RB, 128), lambda i, c: (c, i, 0)),
        out_shape=jax.ShapeDtypeStruct((2, N, 128), jnp.float32),
    )(x, W1r, degT)

    msg1 = _assemble(_spmm2(h1p.reshape(2 * N, 128), srcp, dstp), 2)

    h2p = pl.pallas_call(
        _k2_body,
        grid=(25, 4),
        in_specs=[
            pl.BlockSpec((2, RB, 128), lambda i, c: (0, i, 0)),
            pl.BlockSpec((2, RB, 128), lambda i, c: (0, i, 0)),
            pl.BlockSpec((RB, 2), lambda i, c: (i, 0)),
            pl.BlockSpec((2, 128), lambda i, c: (0, 0)),
            pl.BlockSpec((1, 256, 128), lambda i, c: (c, 0, 0)),
        ],
        out_specs=pl.BlockSpec((1, RB, 128), lambda i, c: (c, i, 0)),
        out_shape=jax.ShapeDtypeStruct((4, N, 128), jnp.float32),
    )(msg1, h1p, degT, b1r, W2r)

    msg2 = _assemble(_spmm4(h2p.reshape(4 * N, 128), srcp, dstp), 4)

    out = pl.pallas_call(
        _k3_body,
        grid=(25,),
        in_specs=[
            pl.BlockSpec((4, RB, 128), lambda i: (0, i, 0)),
            pl.BlockSpec((4, RB, 128), lambda i: (0, i, 0)),
            pl.BlockSpec((RB, 2), lambda i: (i, 0)),
            pl.BlockSpec((4, 128), lambda i: (0, 0)),
        ],
        out_specs=pl.BlockSpec((RB, 512), lambda i: (i, 0)),
        out_shape=jax.ShapeDtypeStruct((N, 512), jnp.float32),
    )(msg2, h2p, degT, b2r)
    return out


# speedup vs baseline: 4.2770x; 1.1071x over previous
import functools
import jax
import jax.numpy as jnp
from jax import lax
from jax.experimental import pallas as pl
from jax.experimental.pallas import tpu as pltpu
import jax.experimental.pallas.tpu_sc as plsc

N = 10000
E = 160000
NC, NS = 2, 16
HALF = N // NC
HPAD = 5120
ROWS_PER_TILE = HPAD // NS
LSINK = 5056
B = 128
NB = 79
EPAD = NS * NB * B
NPAD = 10240
DROWS = NPAD // NS
DEG_NB = 40
DEG_EPAD_HALF = NS * DEG_NB * B
DSINK = 10016

_mesh = plsc.VectorSubcoreMesh(
    core_axis_name="c", subcore_axis_name="s", num_cores=NC, num_subcores=NS
)


@functools.partial(
    pl.kernel,
    out_type=jax.ShapeDtypeStruct((NC, NPAD), jnp.float32),
    mesh=_mesh,
    scratch_types=[
        pltpu.VMEM((DEG_NB, B), jnp.int32),
        pltpu.VMEM((B,), jnp.float32),
        pltpu.VMEM((DROWS,), jnp.float32),
        pltpu.VMEM_SHARED((NPAD,), jnp.float32),
    ],
)
def _deg_kernel(dstd, out, dv, ones_v, zb, acc):
    c = lax.axis_index("c")
    s = lax.axis_index("s")
    one16 = jnp.ones((16,), jnp.float32)
    zero16 = jnp.zeros((16,), jnp.float32)
    for k in range(B // 16):
        ones_v[pl.ds(k * 16, 16)] = one16
    for k in range(DROWS // 16):
        zb[pl.ds(k * 16, 16)] = zero16
    pltpu.sync_copy(zb, acc.at[pl.ds(s * DROWS, DROWS)])
    pltpu.sync_copy(dstd.at[c, s], dv)
    plsc.subcore_barrier()

    def blk(j, _):
        pltpu.sync_copy(ones_v, acc.at[dv.at[j]], add=True)
        return 0

    lax.fori_loop(0, DEG_NB, blk, 0)
    plsc.subcore_barrier()
    pltpu.sync_copy(
        acc.at[pl.ds(s * DROWS, DROWS)],
        out.at[c, pl.ds(s * DROWS, DROWS)],
    )


def _make_spmm(C):
    @functools.partial(
        pl.kernel,
        out_type=jax.ShapeDtypeStruct((C, NC, HPAD, 128), jnp.float32),
        mesh=_mesh,
        scratch_types=[
            pltpu.VMEM((NB, B), jnp.int32),
            pltpu.VMEM((NB, B), jnp.int32),
            pltpu.VMEM((B, 128), jnp.float32),
            pltpu.VMEM((B, 128), jnp.float32),
            pltpu.VMEM((64, 128), jnp.float32),
            pltpu.VMEM_SHARED((HPAD, 128), jnp.float32),
            pltpu.SemaphoreType.DMA,
        ],
    )
    def spmm(htab, srcp, dstp, out, sv, dv, ra, rb, zb, acc, gsem):
        c = lax.axis_index("c")
        s = lax.axis_index("s")
        zero16 = jnp.zeros((16,), jnp.float32)

        def zrow(i, _):
            for k in range(8):
                zb[i, pl.ds(k * 16, 16)] = zero16
            return 0

        lax.fori_loop(0, 64, zrow, 0)

        pltpu.sync_copy(dstp.at[s], dv)
        lo = (c * HALF).astype(jnp.int32)
        sink16 = jnp.full((16,), LSINK, jnp.int32)

        def remap(i, _):
            for k in range(B // 16):
                sl = pl.ds(k * 16, 16)
                d = dv[i, sl]
                dl = d - lo
                ok = (dl >= 0) & (dl < HALF)
                dv[i, sl] = jnp.where(ok, dl, sink16)
            return 0

        lax.fori_loop(0, NB, remap, 0)

        for chunk in range(C):
            pltpu.sync_copy(srcp.at[s], sv)
            off = jnp.int32(chunk * N)

            def shift(i, _):
                for k in range(B // 16):
                    sl = pl.ds(k * 16, 16)
                    sv[i, sl] = sv[i, sl] + off
                return 0

            if chunk:
                lax.fori_loop(0, NB, shift, 0)

            for z in range(ROWS_PER_TILE // 64):
                pltpu.sync_copy(zb, acc.at[pl.ds(s * ROWS_PER_TILE + z * 64, 64)])
            plsc.subcore_barrier()

            pltpu.make_async_copy(htab.at[sv.at[0]], ra, gsem).start()

            def pair(k, _):
                a = 2 * k
                pltpu.make_async_copy(htab.at[sv.at[a]], ra, gsem).wait()
                pltpu.make_async_copy(htab.at[sv.at[a + 1]], rb, gsem).start()
                pltpu.sync_copy(ra, acc.at[dv.at[a]], add=True)
                pltpu.make_async_copy(htab.at[sv.at[a + 1]], rb, gsem).wait()
                pl.when(a + 2 < NB)(
                    lambda: pltpu.make_async_copy(
                        htab.at[sv.at[a + 2]], ra, gsem
                    ).start()
                )
                pltpu.sync_copy(rb, acc.at[dv.at[a + 1]], add=True)
                return 0

            lax.fori_loop(0, NB // 2, pair, 0)
            pltpu.make_async_copy(htab.at[sv.at[NB - 1]], ra, gsem).wait()
            pltpu.sync_copy(ra, acc.at[dv.at[NB - 1]], add=True)
            plsc.subcore_barrier()
            for z in range(ROWS_PER_TILE // 64):
                pltpu.sync_copy(
                    acc.at[pl.ds(s * ROWS_PER_TILE + z * 64, 64)],
                    out.at[chunk, c, pl.ds(s * ROWS_PER_TILE + z * 64, 64)],
                )
            plsc.subcore_barrier()

    return spmm


_spmm2 = _make_spmm(2)
_spmm4 = _make_spmm(4)


def _dinv_of(degT_blk):
    dsum = degT_blk[:, 0:1] + degT_blk[:, 1:2] + 1.0
    return lax.rsqrt(dsum)


def _k1_body(x_ref, w_ref, degT_ref, out_ref):
    dinv = _dinv_of(degT_ref[...])
    h = jnp.dot(x_ref[...], w_ref[0], preferred_element_type=jnp.float32)
    out_ref[0] = h * dinv


def _k2_body(msg_ref, hp_ref, degT_ref, b_ref, w_ref, out_ref):
    dinv = _dinv_of(degT_ref[...])
    parts = []
    for j in range(2):
        t = dinv * (msg_ref[j] + hp_ref[j]) + b_ref[j][None, :]
        parts.append(jnp.maximum(t, 0.0))
    o1 = jnp.concatenate(parts, axis=1)
    h = jnp.dot(o1, w_ref[0], preferred_element_type=jnp.float32)
    out_ref[0] = h * dinv


def _k3_body(msg_ref, hp_ref, degT_ref, b_ref, out_ref):
    dinv = _dinv_of(degT_ref[...])
    parts = []
    for j in range(4):
        parts.append(dinv * (msg_ref[j] + hp_ref[j]) + b_ref[j][None, :])
    out_ref[...] = jnp.concatenate(parts, axis=1)


RB = 400


def _assemble(msgp, C):
    return jnp.concatenate([msgp[:, 0, :HALF], msgp[:, 1, :HALF]], axis=1)


def kernel(x, edge_index, W1, b1, W2, b2):
    ei = edge_index.astype(jnp.int32)
    src, dst = ei[0], ei[1]

    pad = EPAD - E
    srcp = jnp.concatenate([src, jnp.zeros((pad,), jnp.int32)]).reshape(NS, NB, B)
    dstp = jnp.concatenate([dst, jnp.full((pad,), DSINK, jnp.int32)]).reshape(NS, NB, B)
    half = E // 2
    hpad = DEG_EPAD_HALF - half
    dstd = jnp.concatenate(
        [
            dst[:half], jnp.full((hpad,), DSINK, jnp.int32),
            dst[half:], jnp.full((hpad,), DSINK, jnp.int32),
        ]
    ).reshape(NC, NS, DEG_NB, B)

    W1r = W1.reshape(512, 2, 128).transpose(1, 0, 2)
    W2r = W2.reshape(256, 4, 128).transpose(1, 0, 2)
    b1r = b1.reshape(2, 128)
    b2r = b2.reshape(4, 128)

    degp = _deg_kernel(dstd)
    degT = degp.T[:N]

    h1p = pl.pallas_call(
        _k1_body,
        grid=(25, 2),
        in_specs=[
            pl.BlockSpec((RB, 512), lambda i, c: (i, 0)),
            pl.BlockSpec((1, 512, 128), lambda i, c: (c, 0, 0)),
            pl.BlockSpec((RB, 2), lambda i, c: (i, 0)),
        ],
        out_specs=pl.BlockSpec((1, RB, 128), lambda i, c: (c, i, 0)),
        out_shape=jax.ShapeDtypeStruct((2, N, 128), jnp.float32),
    )(x, W1r, degT)

    msg1 = _assemble(_spmm2(h1p.reshape(2 * N, 128), srcp, dstp), 2)

    h2p = pl.pallas_call(
        _k2_body,
        grid=(25, 4),
        in_specs=[
            pl.BlockSpec((2, RB, 128), lambda i, c: (0, i, 0)),
            pl.BlockSpec((2, RB, 128), lambda i, c: (0, i, 0)),
            pl.BlockSpec((RB, 2), lambda i, c: (i, 0)),
            pl.BlockSpec((2, 128), lambda i, c: (0, 0)),
            pl.BlockSpec((1, 256, 128), lambda i, c: (c, 0, 0)),
        ],
        out_specs=pl.BlockSpec((1, RB, 128), lambda i, c: (c, i, 0)),
        out_shape=jax.ShapeDtypeStruct((4, N, 128), jnp.float32),
    )(msg1, h1p, degT, b1r, W2r)

    msg2 = _assemble(_spmm4(h2p.reshape(4 * N, 128), srcp, dstp), 4)

    out = pl.pallas_call(
        _k3_body,
        grid=(25,),
        in_specs=[
            pl.BlockSpec((4, RB, 128), lambda i: (0, i, 0)),
            pl.BlockSpec((4, RB, 128), lambda i: (0, i, 0)),
            pl.BlockSpec((RB, 2), lambda i: (i, 0)),
            pl.BlockSpec((4, 128), lambda i: (0, 0)),
        ],
        out_specs=pl.BlockSpec((RB, 512), lambda i: (i, 0)),
        out_shape=jax.ShapeDtypeStruct((N, 512), jnp.float32),
    )(msg2, h2p, degT, b2r)
    return out


# speedup vs baseline: 4.2776x; 1.0001x over previous
import functools
import jax
import jax.numpy as jnp
from jax import lax
from jax.experimental import pallas as pl
from jax.experimental.pallas import tpu as pltpu
import jax.experimental.pallas.tpu_sc as plsc

N = 10000
E = 160000
NC, NS = 2, 16
HALF = N // NC
HPAD = 5120
ROWS_PER_TILE = HPAD // NS
LSINK = 5056
B = 128
NB = 79
EPAD = NS * NB * B
NPAD = 10240
DROWS = NPAD // NS
DEG_NB = 40
DEG_EPAD_HALF = NS * DEG_NB * B
DSINK = 10016

_mesh = plsc.VectorSubcoreMesh(
    core_axis_name="c", subcore_axis_name="s", num_cores=NC, num_subcores=NS
)


@functools.partial(
    pl.kernel,
    out_type=jax.ShapeDtypeStruct((NC, NPAD), jnp.float32),
    mesh=_mesh,
    scratch_types=[
        pltpu.VMEM((DEG_NB, B), jnp.int32),
        pltpu.VMEM((B,), jnp.float32),
        pltpu.VMEM((DROWS,), jnp.float32),
        pltpu.VMEM_SHARED((NPAD,), jnp.float32),
    ],
)
def _deg_kernel(dstd, out, dv, ones_v, zb, acc):
    c = lax.axis_index("c")
    s = lax.axis_index("s")
    one16 = jnp.ones((16,), jnp.float32)
    zero16 = jnp.zeros((16,), jnp.float32)
    for k in range(B // 16):
        ones_v[pl.ds(k * 16, 16)] = one16
    for k in range(DROWS // 16):
        zb[pl.ds(k * 16, 16)] = zero16
    pltpu.sync_copy(zb, acc.at[pl.ds(s * DROWS, DROWS)])
    pltpu.sync_copy(dstd.at[c, s], dv)
    plsc.subcore_barrier()

    def blk(j, _):
        pltpu.sync_copy(ones_v, acc.at[dv.at[j]], add=True)
        return 0

    lax.fori_loop(0, DEG_NB, blk, 0)
    plsc.subcore_barrier()
    pltpu.sync_copy(
        acc.at[pl.ds(s * DROWS, DROWS)],
        out.at[c, pl.ds(s * DROWS, DROWS)],
    )


def _make_spmm(C):
    @functools.partial(
        pl.kernel,
        out_type=jax.ShapeDtypeStruct((C, NC, HPAD, 128), jnp.float32),
        mesh=_mesh,
        scratch_types=[
            pltpu.VMEM((NB, B), jnp.int32),
            pltpu.VMEM((NB, B), jnp.int32),
            pltpu.VMEM((B, 128), jnp.float32),
            pltpu.VMEM((B, 128), jnp.float32),
            pltpu.VMEM((64, 128), jnp.float32),
            pltpu.VMEM_SHARED((HPAD, 128), jnp.float32),
            pltpu.SemaphoreType.DMA,
            pltpu.SemaphoreType.DMA,
            pltpu.SemaphoreType.DMA,
        ],
    )
    def spmm(htab, srcp, dstp, out, sv, dv, ra, rb, zb, acc, gsem, sa, sb):
        c = lax.axis_index("c")
        s = lax.axis_index("s")
        zero16 = jnp.zeros((16,), jnp.float32)

        def zrow(i, _):
            for k in range(8):
                zb[i, pl.ds(k * 16, 16)] = zero16
            return 0

        lax.fori_loop(0, 64, zrow, 0)

        pltpu.sync_copy(dstp.at[s], dv)
        lo = (c * HALF).astype(jnp.int32)
        sink16 = jnp.full((16,), LSINK, jnp.int32)

        def remap(i, _):
            for k in range(B // 16):
                sl = pl.ds(k * 16, 16)
                d = dv[i, sl]
                dl = d - lo
                ok = (dl >= 0) & (dl < HALF)
                dv[i, sl] = jnp.where(ok, dl, sink16)
            return 0

        lax.fori_loop(0, NB, remap, 0)

        for chunk in range(C):
            pltpu.sync_copy(srcp.at[s], sv)
            off = jnp.int32(chunk * N)

            def shift(i, _):
                for k in range(B // 16):
                    sl = pl.ds(k * 16, 16)
                    sv[i, sl] = sv[i, sl] + off
                return 0

            if chunk:
                lax.fori_loop(0, NB, shift, 0)

            for z in range(ROWS_PER_TILE // 64):
                pltpu.sync_copy(zb, acc.at[pl.ds(s * ROWS_PER_TILE + z * 64, 64)])
            plsc.subcore_barrier()

            pltpu.make_async_copy(htab.at[sv.at[0]], ra, gsem).start()

            def pair(k, _):
                a = 2 * k
                pltpu.make_async_copy(htab.at[sv.at[a]], ra, gsem).wait()
                pltpu.make_async_copy(ra, acc.at[dv.at[a]], sa).start(add=True)
                pl.when(k > 0)(
                    lambda: pltpu.make_async_copy(
                        rb, acc.at[dv.at[a - 1]], sb
                    ).wait()
                )
                pltpu.make_async_copy(htab.at[sv.at[a + 1]], rb, gsem).start()
                pltpu.make_async_copy(htab.at[sv.at[a + 1]], rb, gsem).wait()
                pltpu.make_async_copy(rb, acc.at[dv.at[a + 1]], sb).start(add=True)
                pltpu.make_async_copy(ra, acc.at[dv.at[a]], sa).wait()
                pl.when(a + 2 < NB)(
                    lambda: pltpu.make_async_copy(
                        htab.at[sv.at[a + 2]], ra, gsem
                    ).start()
                )
                return 0

            lax.fori_loop(0, NB // 2, pair, 0)
            pltpu.make_async_copy(htab.at[sv.at[NB - 1]], ra, gsem).wait()
            pltpu.make_async_copy(rb, acc.at[dv.at[NB - 2]], sb).wait()
            pltpu.sync_copy(ra, acc.at[dv.at[NB - 1]], add=True)
            plsc.subcore_barrier()
            for z in range(ROWS_PER_TILE // 64):
                pltpu.sync_copy(
                    acc.at[pl.ds(s * ROWS_PER_TILE + z * 64, 64)],
                    out.at[chunk, c, pl.ds(s * ROWS_PER_TILE + z * 64, 64)],
                )
            plsc.subcore_barrier()

    return spmm


_spmm2 = _make_spmm(2)
_spmm4 = _make_spmm(4)


def _dinv_of(degT_blk):
    dsum = degT_blk[:, 0:1] + degT_blk[:, 1:2] + 1.0
    return lax.rsqrt(dsum)


def _k1_body(x_ref, w_ref, degT_ref, out_ref):
    dinv = _dinv_of(degT_ref[...])
    h = jnp.dot(x_ref[...], w_ref[0], preferred_element_type=jnp.float32)
    out_ref[0] = h * dinv


def _k2_body(msg_ref, hp_ref, degT_ref, b_ref, w_ref, out_ref):
    dinv = _dinv_of(degT_ref[...])
    parts = []
    for j in range(2):
        t = dinv * (msg_ref[j] + hp_ref[j]) + b_ref[j][None, :]
        parts.append(jnp.maximum(t, 0.0))
    o1 = jnp.concatenate(parts, axis=1)
    h = jnp.dot(o1, w_ref[0], preferred_element_type=jnp.float32)
    out_ref[0] = h * dinv


def _k3_body(msg_ref, hp_ref, degT_ref, b_ref, out_ref):
    dinv = _dinv_of(degT_ref[...])
    parts = []
    for j in range(4):
        parts.append(dinv * (msg_ref[j] + hp_ref[j]) + b_ref[j][None, :])
    out_ref[...] = jnp.concatenate(parts, axis=1)


RB = 400


def _assemble(msgp, C):
    return jnp.concatenate([msgp[:, 0, :HALF], msgp[:, 1, :HALF]], axis=1)


def kernel(x, edge_index, W1, b1, W2, b2):
    ei = edge_index.astype(jnp.int32)
    src, dst = ei[0], ei[1]

    pad = EPAD - E
    srcp = jnp.concatenate([src, jnp.zeros((pad,), jnp.int32)]).reshape(NS, NB, B)
    dstp = jnp.concatenate([dst, jnp.full((pad,), DSINK, jnp.int32)]).reshape(NS, NB, B)
    half = E // 2
    hpad = DEG_EPAD_HALF - half
    dstd = jnp.concatenate(
        [
            dst[:half], jnp.full((hpad,), DSINK, jnp.int32),
            dst[half:], jnp.full((hpad,), DSINK, jnp.int32),
        ]
    ).reshape(NC, NS, DEG_NB, B)

    W1r = W1.reshape(512, 2, 128).transpose(1, 0, 2)
    W2r = W2.reshape(256, 4, 128).transpose(1, 0, 2)
    b1r = b1.reshape(2, 128)
    b2r = b2.reshape(4, 128)

    degp = _deg_kernel(dstd)
    degT = degp.T[:N]

    h1p = pl.pallas_call(
        _k1_body,
        grid=(25, 2),
        in_specs=[
            pl.BlockSpec((RB, 512), lambda i, c: (i, 0)),
            pl.BlockSpec((1, 512, 128), lambda i, c: (c, 0, 0)),
            pl.BlockSpec((RB, 2), lambda i, c: (i, 0)),
        ],
        out_specs=pl.BlockSpec((1, RB, 128), lambda i, c: (c, i, 0)),
        out_shape=jax.ShapeDtypeStruct((2, N, 128), jnp.float32),
    )(x, W1r, degT)

    msg1 = _assemble(_spmm2(h1p.reshape(2 * N, 128), srcp, dstp), 2)

    h2p = pl.pallas_call(
        _k2_body,
        grid=(25, 4),
        in_specs=[
            pl.BlockSpec((2, RB, 128), lambda i, c: (0, i, 0)),
            pl.BlockSpec((2, RB, 128), lambda i, c: (0, i, 0)),
            pl.BlockSpec((RB, 2), lambda i, c: (i, 0)),
            pl.BlockSpec((2, 128), lambda i, c: (0, 0)),
            pl.BlockSpec((1, 256, 128), lambda i, c: (c, 0, 0)),
        ],
        out_specs=pl.BlockSpec((1, RB, 128), lambda i, c: (c, i, 0)),
        out_shape=jax.ShapeDtypeStruct((4, N, 128), jnp.float32),
    )(msg1, h1p, degT, b1r, W2r)

    msg2 = _assemble(_spmm4(h2p.reshape(4 * N, 128), srcp, dstp), 4)

    out = pl.pallas_call(
        _k3_body,
        grid=(25,),
        in_specs=[
            pl.BlockSpec((4, RB, 128), lambda i: (0, i, 0)),
            pl.BlockSpec((4, RB, 128), lambda i: (0, i, 0)),
            pl.BlockSpec((RB, 2), lambda i: (i, 0)),
            pl.BlockSpec((4, 128), lambda i: (0, 0)),
        ],
        out_specs=pl.BlockSpec((RB, 512), lambda i: (i, 0)),
        out_shape=jax.ShapeDtypeStruct((N, 512), jnp.float32),
    )(msg2, h2p, degT, b2r)
    return out
